# Initial kernel scaffold; baseline (speedup 1.0000x reference)
import jax
import jax.numpy as jnp
from jax.experimental import pallas as pl


def kernel(inputs, emb_table, W, b):
    raise NotImplementedError("write your pallas kernel here")


# speedup vs baseline: 1.2219x; 1.2219x over previous
import functools

import jax
import jax.numpy as jnp
from jax import lax
from jax.experimental import pallas as pl
from jax.experimental.pallas import tpu as pltpu
from jax.experimental.pallas import tpu_sc as plsc

_NUM_WORKERS = 32
_LANES = 16


def _bag_body(l_half, rows_per_worker, inp_hbm, tbl_hbm, out_hbm,
              idx_v, rows_v, acc_v, sem):
  c = lax.axis_index("c")
  s = lax.axis_index("s")
  wid = s * 2 + c
  base = wid * rows_per_worker
  embed = tbl_hbm.shape[1]
  n_vregs = embed // _LANES

  def row_body(r, carry):
    pltpu.sync_copy(inp_hbm.at[base + r], idx_v)
    cp0 = pltpu.async_copy(tbl_hbm.at[idx_v.at[0]],
                           rows_v.at[pl.ds(0, l_half)], sem)
    cp1 = pltpu.async_copy(tbl_hbm.at[idx_v.at[1]],
                           rows_v.at[pl.ds(l_half, l_half)], sem)
    cp0.wait()
    cp1.wait()

    def red(i, accs):
      cur = list(accs)
      for u in range(4):
        row = i * 4 + u
        for j in range(n_vregs):
          cur[j] = cur[j] + rows_v[row, pl.ds(_LANES * j, _LANES)]
      return tuple(cur)

    zeros = tuple(jnp.zeros((_LANES,), jnp.float32) for _ in range(n_vregs))
    accs = lax.fori_loop(0, (2 * l_half) // 4, red, zeros)
    for j in range(n_vregs):
      acc_v[r, pl.ds(_LANES * j, _LANES)] = accs[j]
    return carry

  lax.fori_loop(0, rows_per_worker, row_body, 0)
  pltpu.sync_copy(acc_v, out_hbm.at[pl.ds(base, rows_per_worker)])


def _bag(idx, emb_table):
  b, _, l_half = idx.shape
  embed = emb_table.shape[1]
  rows_per_worker = b // _NUM_WORKERS
  mesh = plsc.VectorSubcoreMesh(core_axis_name="c", subcore_axis_name="s")
  return pl.kernel(
      functools.partial(_bag_body, l_half, rows_per_worker),
      out_type=jax.ShapeDtypeStruct((b, embed), jnp.float32),
      mesh=mesh,
      compiler_params=pltpu.CompilerParams(use_tc_tiling_on_sc=False),
      scratch_types=[
          pltpu.VMEM((2, l_half), jnp.int32),
          pltpu.VMEM((2 * l_half, embed), jnp.float32),
          pltpu.VMEM((rows_per_worker, embed), jnp.float32),
          pltpu.SemaphoreType.DMA,
      ],
  )(idx, emb_table)


def _proj_body(x_ref, w_ref, b_ref, o_ref):
  o_ref[...] = lax.dot_general(
      x_ref[...], w_ref[...],
      (((1,), (1,)), ((), ())),
      preferred_element_type=jnp.float32) + b_ref[...]


def _proj(x, w, bias):
  b, embed = x.shape
  v = w.shape[0]
  nb = 512
  return pl.pallas_call(
      _proj_body,
      grid=(pl.cdiv(v, nb),),
      in_specs=[
          pl.BlockSpec((b, embed), lambda i: (0, 0)),
          pl.BlockSpec((nb, embed), lambda i: (i, 0)),
          pl.BlockSpec((1, nb), lambda i: (0, i)),
      ],
      out_specs=pl.BlockSpec((b, nb), lambda i: (0, i)),
      out_shape=jax.ShapeDtypeStruct((b, v), jnp.float32),
      compiler_params=pltpu.CompilerParams(
          dimension_semantics=("arbitrary",)),
  )(x, w, bias)


def kernel(inputs, emb_table, W, b):
  bsz, l = inputs.shape
  idx = inputs.astype(jnp.int32).reshape(bsz, 2, l // 2)
  bag = _bag(idx, emb_table)
  return _proj(bag, W, b.reshape(1, -1))


# speedup vs baseline: 1.3705x; 1.1216x over previous
import functools

import jax
import jax.numpy as jnp
from jax import lax
from jax.experimental import pallas as pl
from jax.experimental.pallas import tpu as pltpu
from jax.experimental.pallas import tpu_sc as plsc

_NUM_WORKERS = 32
_LANES = 16


def _bag_body(l_half, rows_per_worker, inp_hbm, tbl_hbm, out_hbm,
              idx_v, rows_v, acc_v, sem):
  c = lax.axis_index("c")
  s = lax.axis_index("s")
  wid = s * 2 + c
  base = wid * rows_per_worker
  embed = tbl_hbm.shape[1]
  n_vregs = embed // _LANES

  def row_body(r, carry):
    pltpu.sync_copy(inp_hbm.at[base + r], idx_v)
    cp0 = pltpu.async_copy(tbl_hbm.at[idx_v.at[0]],
                           rows_v.at[pl.ds(0, l_half)], sem)
    cp1 = pltpu.async_copy(tbl_hbm.at[idx_v.at[1]],
                           rows_v.at[pl.ds(l_half, l_half)], sem)
    cp0.wait()
    cp1.wait()

    def red(i, accs):
      cur = list(accs)
      for u in range(4):
        row = i * 4 + u
        for j in range(n_vregs):
          cur[j] = cur[j] + rows_v[row, pl.ds(_LANES * j, _LANES)]
      return tuple(cur)

    zeros = tuple(jnp.zeros((_LANES,), jnp.float32) for _ in range(n_vregs))
    accs = lax.fori_loop(0, (2 * l_half) // 4, red, zeros)
    for j in range(n_vregs):
      acc_v[r, pl.ds(_LANES * j, _LANES)] = accs[j]
    return carry

  lax.fori_loop(0, rows_per_worker, row_body, 0)
  pltpu.sync_copy(acc_v, out_hbm.at[pl.ds(base, rows_per_worker)])


def _bag(idx, emb_table):
  b, _, l_half = idx.shape
  embed = emb_table.shape[1]
  rows_per_worker = b // _NUM_WORKERS
  mesh = plsc.VectorSubcoreMesh(core_axis_name="c", subcore_axis_name="s")
  return pl.kernel(
      functools.partial(_bag_body, l_half, rows_per_worker),
      out_type=jax.ShapeDtypeStruct((b, embed), jnp.float32),
      mesh=mesh,
      compiler_params=pltpu.CompilerParams(use_tc_tiling_on_sc=False),
      scratch_types=[
          pltpu.VMEM((2, l_half), jnp.int32),
          pltpu.VMEM((2 * l_half, embed), jnp.float32),
          pltpu.VMEM((rows_per_worker, embed), jnp.float32),
          pltpu.SemaphoreType.DMA,
      ],
  )(idx, emb_table)


def _proj_body(x_ref, w_ref, b_ref, o_ref):
  o_ref[...] = lax.dot_general(
      x_ref[...], w_ref[...],
      (((1,), (1,)), ((), ())),
      preferred_element_type=jnp.float32) + b_ref[...]


def _proj(x, w, bias):
  b, embed = x.shape
  v = w.shape[0]
  nb = 2048
  return pl.pallas_call(
      _proj_body,
      grid=(pl.cdiv(v, nb),),
      in_specs=[
          pl.BlockSpec((b, embed), lambda i: (0, 0)),
          pl.BlockSpec((nb, embed), lambda i: (i, 0)),
          pl.BlockSpec((1, nb), lambda i: (0, i)),
      ],
      out_specs=pl.BlockSpec((b, nb), lambda i: (0, i)),
      out_shape=jax.ShapeDtypeStruct((b, v), jnp.float32),
      compiler_params=pltpu.CompilerParams(
          dimension_semantics=("arbitrary",)),
  )(x, w, bias)


def kernel(inputs, emb_table, W, b):
  bsz, l = inputs.shape
  idx = inputs.astype(jnp.int32).reshape(bsz, 2, l // 2)
  bag = _bag(idx, emb_table)
  return _proj(bag, W, b.reshape(1, -1))


# speedup vs baseline: 1.3778x; 1.0053x over previous
import functools

import jax
import jax.numpy as jnp
from jax import lax
from jax.experimental import pallas as pl
from jax.experimental.pallas import tpu as pltpu
from jax.experimental.pallas import tpu_sc as plsc

_NUM_WORKERS = 32
_LANES = 16


def _bag_body(l_half, rows_per_worker, inp_hbm, tbl_hbm, out_hbm,
              idx_v, rows_v, acc_v, sem):
  c = lax.axis_index("c")
  s = lax.axis_index("s")
  wid = s * 2 + c
  base = wid * rows_per_worker
  embed = tbl_hbm.shape[1]
  n_vregs = embed // _LANES

  def row_body(r, carry):
    pltpu.sync_copy(inp_hbm.at[base + r], idx_v)
    cp0 = pltpu.async_copy(tbl_hbm.at[idx_v.at[0]],
                           rows_v.at[pl.ds(0, l_half)], sem)
    cp1 = pltpu.async_copy(tbl_hbm.at[idx_v.at[1]],
                           rows_v.at[pl.ds(l_half, l_half)], sem)
    cp0.wait()
    cp1.wait()

    def red(i, accs):
      cur = list(accs)
      for u in range(4):
        row = i * 4 + u
        for j in range(n_vregs):
          cur[j] = cur[j] + rows_v[row, pl.ds(_LANES * j, _LANES)]
      return tuple(cur)

    zeros = tuple(jnp.zeros((_LANES,), jnp.float32) for _ in range(n_vregs))
    accs = lax.fori_loop(0, (2 * l_half) // 4, red, zeros)
    for j in range(n_vregs):
      acc_v[r, pl.ds(_LANES * j, _LANES)] = accs[j]
    return carry

  lax.fori_loop(0, rows_per_worker, row_body, 0)
  pltpu.sync_copy(acc_v, out_hbm.at[pl.ds(base, rows_per_worker)])


def _bag(idx, emb_table):
  b, _, l_half = idx.shape
  embed = emb_table.shape[1]
  rows_per_worker = b // _NUM_WORKERS
  mesh = plsc.VectorSubcoreMesh(core_axis_name="c", subcore_axis_name="s")
  return pl.kernel(
      functools.partial(_bag_body, l_half, rows_per_worker),
      out_type=jax.ShapeDtypeStruct((b, embed), jnp.float32),
      mesh=mesh,
      compiler_params=pltpu.CompilerParams(use_tc_tiling_on_sc=False),
      scratch_types=[
          pltpu.VMEM((2, l_half), jnp.int32),
          pltpu.VMEM((2 * l_half, embed), jnp.float32),
          pltpu.VMEM((rows_per_worker, embed), jnp.float32),
          pltpu.SemaphoreType.DMA,
      ],
  )(idx, emb_table)


def _proj_body(x_ref, w_ref, b_ref, o_ref):
  o_ref[...] = lax.dot_general(
      x_ref[...], w_ref[...],
      (((1,), (1,)), ((), ())),
      preferred_element_type=jnp.float32) + b_ref[...]


def _proj(x, w, bias):
  b, embed = x.shape
  v = w.shape[0]
  nb = 4096
  return pl.pallas_call(
      _proj_body,
      grid=(pl.cdiv(v, nb),),
      in_specs=[
          pl.BlockSpec((b, embed), lambda i: (0, 0)),
          pl.BlockSpec((nb, embed), lambda i: (i, 0)),
          pl.BlockSpec((1, nb), lambda i: (0, i)),
      ],
      out_specs=pl.BlockSpec((b, nb), lambda i: (0, i)),
      out_shape=jax.ShapeDtypeStruct((b, v), jnp.float32),
      compiler_params=pltpu.CompilerParams(
          dimension_semantics=("arbitrary",)),
  )(x, w, bias)


def kernel(inputs, emb_table, W, b):
  bsz, l = inputs.shape
  idx = inputs.astype(jnp.int32).reshape(bsz, 2, l // 2)
  bag = _bag(idx, emb_table)
  return _proj(bag, W, b.reshape(1, -1))
